# baseline (speedup 1.0000x reference)
import functools

import jax
import jax.numpy as jnp
from jax import lax
from jax.experimental import pallas as pl
from jax.experimental.pallas import tpu as pltpu
from jax.experimental.pallas import tpu_sc as plsc

HIDDEN = 768
REL = 53
NUM_BAGS = 16
TOTAL = 32768
BAG = TOTAL // NUM_BAGS
LANES = 16


def _score_kernel(x_ref, k_ref, rel_ref, bias_ref, s_ref):
    xc = x_ref[...]
    logits = jnp.dot(xc, rel_ref[...],
                     preferred_element_type=jnp.float32) + bias_ref[...]
    m = jnp.max(logits, axis=1, keepdims=True)
    e = jnp.exp(logits - m)
    p = e / jnp.sum(e, axis=1, keepdims=True)
    s_ref[...] = jnp.sum(p * k_ref[...], axis=1, keepdims=True)


def _final_kernel(rows_ref, rel_ref, bias_ref, out_ref):
    fl = jnp.dot(rows_ref[...], rel_ref[...],
                 preferred_element_type=jnp.float32) + bias_ref[...]
    fm = jnp.max(fl, axis=1, keepdims=True)
    fe = jnp.exp(fl - fm)
    out_ref[...] = fe / jnp.sum(fe, axis=1, keepdims=True)


def _sc_select(scores, x):
    mesh = plsc.VectorSubcoreMesh(core_axis_name="c", subcore_axis_name="s")

    @functools.partial(
        pl.kernel,
        mesh=mesh,
        out_type=jax.ShapeDtypeStruct((NUM_BAGS, HIDDEN), jnp.float32),
        scratch_types=[
            pltpu.VMEM((BAG,), jnp.float32),
            pltpu.VMEM((1, HIDDEN), jnp.float32),
        ],
    )
    def select(scores_hbm, x_hbm, out_hbm, sc_v, row_v):
        wid = lax.axis_index("s") * 2 + lax.axis_index("c")

        @pl.when(wid < NUM_BAGS)
        def _():
            pltpu.sync_copy(scores_hbm.at[wid], sc_v)
            lane = lax.iota(jnp.int32, LANES)

            def body(t, carry):
                m, mi = carry
                base = t * LANES
                v = sc_v[pl.ds(base, LANES)]
                cmp = v > m
                return (jnp.where(cmp, v, m),
                        jnp.where(cmp, lane + base, mi))

            m0 = jnp.full((LANES,), -jnp.inf, jnp.float32)
            i0 = jnp.zeros((LANES,), jnp.int32)
            m, mi = lax.fori_loop(0, BAG // LANES, body, (m0, i0))

            best = m[0]
            j = mi[0]
            for l in range(1, LANES):
                v = m[l]
                idx = mi[l]
                take = (v > best) | ((v == best) & (idx < j))
                best = jnp.where(take, v, best)
                j = jnp.where(take, idx, j)
            pltpu.sync_copy(x_hbm.at[pl.ds(wid * BAG + j, 1)], row_v)
            pltpu.sync_copy(row_v, out_hbm.at[pl.ds(wid, 1)])

    return select(scores, x)


@jax.jit
def _selector(x, knowledge, rel_mat, bias2d):
    scores = pl.pallas_call(
        _score_kernel,
        grid=(NUM_BAGS,),
        in_specs=[
            pl.BlockSpec((BAG, HIDDEN), lambda i: (i, 0)),
            pl.BlockSpec((BAG, REL), lambda i: (i, 0)),
            pl.BlockSpec((HIDDEN, REL), lambda i: (0, 0)),
            pl.BlockSpec((1, REL), lambda i: (0, 0)),
        ],
        out_specs=pl.BlockSpec((BAG, 1), lambda i: (i, 0)),
        out_shape=jax.ShapeDtypeStruct((TOTAL, 1), jnp.float32),
    )(x, knowledge, rel_mat, bias2d)

    rows = _sc_select(scores.reshape(NUM_BAGS, BAG), x)

    return pl.pallas_call(
        _final_kernel,
        in_specs=[
            pl.BlockSpec((NUM_BAGS, HIDDEN), lambda: (0, 0)),
            pl.BlockSpec((HIDDEN, REL), lambda: (0, 0)),
            pl.BlockSpec((1, REL), lambda: (0, 0)),
        ],
        out_specs=pl.BlockSpec((NUM_BAGS, REL), lambda: (0, 0)),
        out_shape=jax.ShapeDtypeStruct((NUM_BAGS, REL), jnp.float32),
    )(rows, rel_mat, bias2d)


def kernel(x, scope, knowledge, rel_mat, bias):
    del scope
    out = _selector(x, knowledge, rel_mat, bias.reshape(1, REL))
    return out, rel_mat


# speedup vs baseline: 1.1437x; 1.1437x over previous
import functools

import jax
import jax.numpy as jnp
from jax import lax
from jax.experimental import pallas as pl
from jax.experimental.pallas import tpu as pltpu
from jax.experimental.pallas import tpu_sc as plsc

HIDDEN = 768
REL = 53
NUM_BAGS = 16
TOTAL = 32768
BAG = TOTAL // NUM_BAGS


def _sweep_kernel(x_ref, k_ref, rel_ref, bias_ref, idx_ref, best_ref):
    b = pl.program_id(0)
    xc = x_ref[...]
    logits = jnp.dot(xc, rel_ref[...],
                     preferred_element_type=jnp.float32) + bias_ref[...]
    m = jnp.max(logits, axis=1, keepdims=True)
    e = jnp.exp(logits - m)
    p = e / jnp.sum(e, axis=1, keepdims=True)
    score = jnp.sum(p * k_ref[...], axis=1, keepdims=True)

    lm = jnp.max(score)
    ids = lax.broadcasted_iota(jnp.int32, (BAG, 1), 0)
    lj = jnp.min(jnp.where(score == lm, ids, BAG))
    idx_ref[b] = b * BAG + lj
    best_ref[0] = lm


def _final_kernel(rows_ref, rel_ref, bias_ref, out_ref):
    fl = jnp.dot(rows_ref[...], rel_ref[...],
                 preferred_element_type=jnp.float32) + bias_ref[...]
    fm = jnp.max(fl, axis=1, keepdims=True)
    fe = jnp.exp(fl - fm)
    out_ref[...] = fe / jnp.sum(fe, axis=1, keepdims=True)


def _sc_gather(idx, x):
    mesh = plsc.VectorSubcoreMesh(core_axis_name="c", subcore_axis_name="s")

    @functools.partial(
        pl.kernel,
        mesh=mesh,
        out_type=jax.ShapeDtypeStruct((NUM_BAGS, HIDDEN), jnp.float32),
        scratch_types=[
            pltpu.VMEM((NUM_BAGS,), jnp.int32),
            pltpu.VMEM((NUM_BAGS, HIDDEN), jnp.float32),
            pltpu.SemaphoreType.DMA,
        ],
    )
    def gather(idx_hbm, x_hbm, out_hbm, idx_v, rows_v, sem):
        wid = lax.axis_index("s") * 2 + lax.axis_index("c")

        @pl.when(wid == 0)
        def _():
            pltpu.sync_copy(idx_hbm, idx_v)
            pltpu.async_copy(x_hbm.at[idx_v], rows_v, sem).wait()
            pltpu.sync_copy(rows_v, out_hbm)

    return gather(idx, x)


@jax.jit
def _selector(x, knowledge, rel_mat, bias2d):
    idx, _ = pl.pallas_call(
        _sweep_kernel,
        grid=(NUM_BAGS,),
        in_specs=[
            pl.BlockSpec((BAG, HIDDEN), lambda i: (i, 0)),
            pl.BlockSpec((BAG, REL), lambda i: (i, 0)),
            pl.BlockSpec((HIDDEN, REL), lambda i: (0, 0)),
            pl.BlockSpec((1, REL), lambda i: (0, 0)),
        ],
        out_specs=[
            pl.BlockSpec(memory_space=pltpu.MemorySpace.SMEM),
            pl.BlockSpec(memory_space=pltpu.MemorySpace.SMEM),
        ],
        out_shape=[
            jax.ShapeDtypeStruct((NUM_BAGS,), jnp.int32),
            jax.ShapeDtypeStruct((1,), jnp.float32),
        ],
    )(x, knowledge, rel_mat, bias2d)

    rows = _sc_gather(idx, x)

    return pl.pallas_call(
        _final_kernel,
        in_specs=[
            pl.BlockSpec((NUM_BAGS, HIDDEN), lambda: (0, 0)),
            pl.BlockSpec((HIDDEN, REL), lambda: (0, 0)),
            pl.BlockSpec((1, REL), lambda: (0, 0)),
        ],
        out_specs=pl.BlockSpec((NUM_BAGS, REL), lambda: (0, 0)),
        out_shape=jax.ShapeDtypeStruct((NUM_BAGS, REL), jnp.float32),
    )(rows, rel_mat, bias2d)


def kernel(x, scope, knowledge, rel_mat, bias):
    del scope
    out = _selector(x, knowledge, rel_mat, bias.reshape(1, REL))
    return out, rel_mat
